# baseline (speedup 1.0000x reference)
import functools

import jax
import jax.numpy as jnp
from jax import lax
from jax.experimental import pallas as pl
from jax.experimental.pallas import tpu as pltpu
from jax.experimental.pallas import tpu_sc as plsc

_N_TAB = 49
_D = 128
_C = 128
_NBUF = 5
_PREF = 3
_NC = 2
_NS = 16
_NW = _NC * _NS


def _sc_body(nsteps, seq_hbm, idx_hbm, tab_hbm, out_hbm,
             tab_sh, idx_v, seq_v,
             isems, ssems, gsems, osems):
    cid = lax.axis_index("c")
    sid = lax.axis_index("s")
    wid = sid * _NC + cid
    rows_per_w = nsteps * _C
    w_base = wid * rows_per_w

    @pl.when(sid == 0)
    def _():
        pltpu.sync_copy(tab_hbm, tab_sh)
    plsc.subcore_barrier()

    def issue_in(g, b):
        base = (g * _NW + wid) * _C
        pltpu.async_copy(idx_hbm.at[pl.ds(base, _C)], idx_v.at[b], isems.at[b])
        pltpu.async_copy(seq_hbm.at[pl.ds(base, _C)], seq_v.at[b], ssems.at[b])

    def wait_in(b):
        pltpu.make_async_copy(idx_hbm.at[pl.ds(0, _C)], idx_v.at[b],
                              isems.at[b]).wait()
        pltpu.make_async_copy(seq_hbm.at[pl.ds(0, _C)], seq_v.at[b],
                              ssems.at[b]).wait()

    def prep_gather(b):
        wait_in(b)
        for i in range(_C // 16):
            v = idx_v[b, pl.ds(i * 16, 16)]
            idx_v[b, pl.ds(i * 16, 16)] = jnp.where(v == -1, _N_TAB - 1, v)
        pltpu.async_copy(tab_sh.at[idx_v.at[b]], seq_v.at[b],
                         gsems.at[b], add=True)

    def wait_gather(b):
        pltpu.make_async_copy(tab_sh.at[idx_v.at[b]], seq_v.at[b],
                              gsems.at[b]).wait()

    for b in range(_PREF):
        issue_in(b, b)
    prep_gather(0)

    def group(grp, carry):
        for b in range(_NBUF):
            g = grp * _NBUF + b
            wait_gather(b)
            pltpu.async_copy(
                seq_v.at[b],
                out_hbm.at[pl.ds((g * _NW + wid) * _C, _C)],
                osems.at[b])
            nb1 = (b + 1) % _NBUF

            @pl.when(g + 1 < nsteps)
            def _():
                prep_gather(nb1)

            nbr = (b + _PREF) % _NBUF

            @pl.when(g + _PREF < nsteps)
            def _():
                @pl.when(g >= _NBUF - _PREF)
                def _():
                    pltpu.make_async_copy(
                        seq_v.at[nbr], out_hbm.at[pl.ds(0, _C)],
                        osems.at[nbr]).wait()
                issue_in(g + _PREF, nbr)
        return carry

    lax.fori_loop(0, nsteps // _NBUF, group, 0)

    for b in range(_NBUF):
        pltpu.make_async_copy(seq_v.at[b], out_hbm.at[pl.ds(0, _C)],
                              osems.at[b]).wait()


def kernel(seq, time_stamp, time_embed):
    B, L, D = seq.shape
    n = B * L
    seq2 = seq.reshape(n, D)
    idx = time_stamp.reshape(-1).astype(jnp.int32)
    nsteps = n // (_NW * _C)
    mesh = plsc.VectorSubcoreMesh(core_axis_name="c", subcore_axis_name="s")
    out = pl.kernel(
        functools.partial(_sc_body, nsteps),
        out_type=jax.ShapeDtypeStruct((n, D), jnp.float32),
        mesh=mesh,
        scratch_types=[
            pltpu.VMEM_SHARED((_N_TAB, _D), jnp.float32),
            pltpu.VMEM((_NBUF, _C), jnp.int32),
            pltpu.VMEM((_NBUF, _C, _D), jnp.float32),
            pltpu.SemaphoreType.DMA((_NBUF,)),
            pltpu.SemaphoreType.DMA((_NBUF,)),
            pltpu.SemaphoreType.DMA((_NBUF,)),
            pltpu.SemaphoreType.DMA((_NBUF,)),
        ],
    )(seq2, idx, time_embed)
    return out.reshape(B, L, D)
